# baseline (speedup 1.0000x reference)
import jax
import jax.numpy as jnp
from jax import lax
from jax.experimental import pallas as pl

NUM_CLASSES = 1000
BR = 128


def _onehot_body(x_ref, o_ref):
    idx = x_ref[...]
    iota = lax.broadcasted_iota(jnp.int32, (BR, x_ref.shape[1], NUM_CLASSES), 2)
    o_ref[...] = (idx[:, :, None] == iota).astype(jnp.int32)


def kernel(x1):
    B, C = x1.shape
    out = pl.pallas_call(
        _onehot_body,
        grid=(B // BR,),
        in_specs=[pl.BlockSpec((BR, C), lambda i: (i, 0))],
        out_specs=pl.BlockSpec((BR, C, NUM_CLASSES), lambda i: (i, 0, 0)),
        out_shape=jax.ShapeDtypeStruct((B, C, NUM_CLASSES), jnp.int32),
    )(x1)
    return out
